# baseline (speedup 1.0000x reference)
import functools

import jax
import jax.numpy as jnp
from jax import lax
from jax.experimental import pallas as pl
from jax.experimental.pallas import tpu as pltpu
from jax.experimental.pallas import tpu_sc as plsc

B = 16384
C = 100
N = 1000000

_NC = 2
_NS = 16
_NW = _NC * _NS
_BPW = B // _NW

_K = 16


def _gather_body(conf_hbm, idx_hbm, out_hbm, idx_v, rows_v, sem):
    wid = lax.axis_index("s") * _NC + lax.axis_index("c")
    base = wid * _BPW
    pltpu.async_copy(idx_hbm.at[pl.ds(base, _BPW)], idx_v, sem).wait()

    def chunk(c, carry):
        r0 = c * _K
        v = idx_v[pl.ds(r0, _K)]
        cps = []
        for j in range(_K):
            i = v[j]
            cp = pltpu.make_async_copy(
                conf_hbm.at[pl.ds(i, 1)], rows_v.at[pl.ds(r0 + j, 1)], sem
            )
            cp.start()
            cps.append(cp)
        for cp in cps:
            cp.wait()
        return carry

    lax.fori_loop(0, _BPW // _K, chunk, 0)
    pltpu.sync_copy(rows_v, out_hbm.at[pl.ds(base, _BPW)])


_gather = functools.partial(
    pl.kernel,
    mesh=plsc.VectorSubcoreMesh(core_axis_name="c", subcore_axis_name="s"),
    out_type=jax.ShapeDtypeStruct((B, C), jnp.float32),
    scratch_types=[
        pltpu.VMEM((_BPW,), jnp.int32),
        pltpu.VMEM((_BPW, C), jnp.float32),
        pltpu.SemaphoreType.DMA,
    ],
)(_gather_body)


_COLS = 512
_GRID = B // _COLS


def _loss_body(out_ref, tgt_ref, acc_ref):
    i = pl.program_id(0)
    x = out_ref[...]
    t = tgt_ref[...].T
    m = jnp.max(x, axis=0, keepdims=True)
    e = jnp.exp(x - m)
    p = e / jnp.sum(e, axis=0, keepdims=True)
    d = p - t
    s = jnp.sum(d * d)

    @pl.when(i == 0)
    def _init():
        acc_ref[0, 0] = 0.0

    acc_ref[0, 0] += s

    @pl.when(i == _GRID - 1)
    def _finish():
        acc_ref[0, 0] = acc_ref[0, 0] / jnp.float32(B * C)


_loss = pl.pallas_call(
    _loss_body,
    grid=(_GRID,),
    in_specs=[
        pl.BlockSpec((C, _COLS), lambda i: (0, i)),
        pl.BlockSpec((_COLS, C), lambda i: (i, 0)),
    ],
    out_specs=pl.BlockSpec(memory_space=pltpu.SMEM),
    out_shape=jax.ShapeDtypeStruct((1, 1), jnp.float32),
)


def kernel(outputs, index, confidence):
    target = _gather(confidence, index)
    loss = _loss(outputs.T, target)
    return loss[0, 0]


# speedup vs baseline: 1.0238x; 1.0238x over previous
import functools

import jax
import jax.numpy as jnp
from jax import lax
from jax.experimental import pallas as pl
from jax.experimental.pallas import tpu as pltpu
from jax.experimental.pallas import tpu_sc as plsc

B = 16384
C = 100
N = 1000000

_NC = 2
_NS = 16
_NW = _NC * _NS
_BPW = B // _NW

_K = 16


def _gather_body(conf_hbm, idx_hbm, out_hbm, idx_v, rows_v, sem):
    wid = lax.axis_index("s") * _NC + lax.axis_index("c")
    base = wid * _BPW
    pltpu.async_copy(idx_hbm.at[pl.ds(base, _BPW)], idx_v, sem).wait()

    def fire(r0):
        v = idx_v[pl.ds(r0, _K)]
        for j in range(_K):
            pltpu.make_async_copy(
                conf_hbm.at[pl.ds(v[j], 1)], rows_v.at[pl.ds(r0 + j, 1)], sem
            ).start()

    def drain(r0):
        v = idx_v[pl.ds(r0, _K)]
        for j in range(_K):
            pltpu.make_async_copy(
                conf_hbm.at[pl.ds(v[j], 1)], rows_v.at[pl.ds(r0 + j, 1)], sem
            ).wait()

    _NCH = _BPW // _K
    fire(0)

    def chunk(c, carry):
        fire((c + 1) * _K)
        drain(c * _K)
        return carry

    lax.fori_loop(0, _NCH - 1, chunk, 0)
    drain((_NCH - 1) * _K)
    pltpu.sync_copy(rows_v, out_hbm.at[pl.ds(base, _BPW)])


_gather = functools.partial(
    pl.kernel,
    mesh=plsc.VectorSubcoreMesh(core_axis_name="c", subcore_axis_name="s"),
    out_type=jax.ShapeDtypeStruct((B, C), jnp.float32),
    scratch_types=[
        pltpu.VMEM((_BPW,), jnp.int32),
        pltpu.VMEM((_BPW, C), jnp.float32),
        pltpu.SemaphoreType.DMA,
    ],
)(_gather_body)


_COLS = 512
_GRID = B // _COLS


def _loss_body(out_ref, tgt_ref, acc_ref):
    i = pl.program_id(0)
    x = out_ref[...]
    t = tgt_ref[...].T
    m = jnp.max(x, axis=0, keepdims=True)
    e = jnp.exp(x - m)
    p = e / jnp.sum(e, axis=0, keepdims=True)
    d = p - t
    s = jnp.sum(d * d)

    @pl.when(i == 0)
    def _init():
        acc_ref[0, 0] = 0.0

    acc_ref[0, 0] += s

    @pl.when(i == _GRID - 1)
    def _finish():
        acc_ref[0, 0] = acc_ref[0, 0] / jnp.float32(B * C)


_loss = pl.pallas_call(
    _loss_body,
    grid=(_GRID,),
    in_specs=[
        pl.BlockSpec((C, _COLS), lambda i: (0, i)),
        pl.BlockSpec((_COLS, C), lambda i: (i, 0)),
    ],
    out_specs=pl.BlockSpec(memory_space=pltpu.SMEM),
    out_shape=jax.ShapeDtypeStruct((1, 1), jnp.float32),
)


def kernel(outputs, index, confidence):
    target = _gather(confidence, index)
    loss = _loss(outputs.T, target)
    return loss[0, 0]
